# baseline (speedup 1.0000x reference)
import functools

import jax
import jax.numpy as jnp
from jax import lax
from jax.experimental import pallas as pl
from jax.experimental.pallas import tpu as pltpu
from jax.experimental.pallas import tpu_sc as plsc

B = 16
AB = 20000
V = 12000
NW = 32
NPW = 384
VP = NW * NPW
NTAIL = V - (NW - 1) * NPW
CH = 128
IDX_ROWS = NPW // CH
ROWS = B * NPW
DROWS = 4 * B * IDX_ROWS
NCHUNK = NPW // 16


def _body(scores_hbm, deltas_hbm, anchors_hbm, idx_hbm, out_hbm,
          idxv, gs, gd, ax, sb, db, ab, ob, sem):
    nc = 2
    w = lax.axis_index("s") * nc + lax.axis_index("c")
    vbase = w * NPW

    with jax.named_scope("ph_idx"):
        for k in range(IDX_ROWS):
            pltpu.sync_copy(idx_hbm.at[pl.ds(vbase + k * CH, CH)], idxv.at[k])

    def build(c, _):
        r = c // 8
        co = r * CH + (c % 8) * 16
        vch = idxv[r, pl.ds((c % 8) * 16, 16)]
        ax[pl.ds(co, 16)] = vch
        ax[pl.ds(NPW + co, 16)] = vch + AB
        ax[pl.ds(2 * NPW + co, 16)] = vch + 2 * AB
        ax[pl.ds(3 * NPW + co, 16)] = vch + 3 * AB
        for b in range(B):
            p = (3 * b) * CH + co
            gs[pl.ds(p, 16)] = vch + b * AB
            vb = vch + b * (4 * AB)
            p4 = (3 * b * 4) * CH + co
            gd[pl.ds(p4, 16)] = vb
            gd[pl.ds(p4 + 3 * CH, 16)] = vb + AB
            gd[pl.ds(p4 + 6 * CH, 16)] = vb + 2 * AB
            gd[pl.ds(p4 + 9 * CH, 16)] = vb + 3 * AB
        return _
    with jax.named_scope("ph_build"):
        lax.fori_loop(0, NCHUNK, build, None)

    with jax.named_scope("ph_fire"):
        pltpu.async_copy(scores_hbm.at[gs], sb, sem)
        pltpu.async_copy(deltas_hbm.at[gd], db, sem)
        pltpu.async_copy(anchors_hbm.at[ax], ab, sem)

    with jax.named_scope("ph_drain"):
        pltpu.make_async_copy(scores_hbm.at[pl.ds(0, ROWS)], sb, sem).wait()
        pltpu.make_async_copy(deltas_hbm.at[pl.ds(0, 4 * ROWS)], db, sem).wait()
        pltpu.make_async_copy(scores_hbm.at[pl.ds(0, 4 * NPW)], ab, sem).wait()

    def compute(c, _):
        x = ab[pl.ds(c * 16, 16)]
        y = ab[pl.ds(NPW + c * 16, 16)]
        ww = ab[pl.ds(2 * NPW + c * 16, 16)]
        hh = ab[pl.ds(3 * NPW + c * 16, 16)]
        for b in range(B):
            p0 = b * NPW + c * 16
            sl = pl.ds(p0, 16)
            dbase = 4 * b * NPW + c * 16
            d0 = db[pl.ds(dbase, 16)]
            d1 = db[pl.ds(dbase + NPW, 16)]
            d2 = db[pl.ds(dbase + 2 * NPW, 16)]
            d3 = db[pl.ds(dbase + 3 * NPW, 16)]
            ob[sl] = sb[sl]
            ob[pl.ds(ROWS + p0, 16)] = x + d0 * ww
            ob[pl.ds(2 * ROWS + p0, 16)] = y + d1 * hh
            ob[pl.ds(3 * ROWS + p0, 16)] = ww * jnp.exp(d2)
            ob[pl.ds(4 * ROWS + p0, 16)] = hh * jnp.exp(d3)
        return _
    with jax.named_scope("ph_compute"):
        lax.fori_loop(0, NCHUNK, compute, None)

    with jax.named_scope("ph_wb"):
        @pl.when(w < NW - 1)
        def _full():
            def wb(j, _):
                pltpu.async_copy(ob.at[pl.ds(j * NPW, NPW)],
                                 out_hbm.at[pl.ds(j * V + vbase, NPW)], sem)
                return _
            lax.fori_loop(0, 5 * B, wb, None)
            pltpu.make_async_copy(out_hbm.at[pl.ds(0, 5 * ROWS)], ob, sem).wait()

        @pl.when(w == NW - 1)
        def _tail():
            def wb(j, _):
                pltpu.async_copy(ob.at[pl.ds(j * NPW, NTAIL)],
                                 out_hbm.at[pl.ds(j * V + vbase, NTAIL)], sem)
                return _
            lax.fori_loop(0, 5 * B, wb, None)
            for j in range(5 * B):
                pltpu.make_async_copy(out_hbm.at[pl.ds(0, NTAIL)],
                                      ob.at[pl.ds(j * NPW, NTAIL)], sem).wait()


@jax.jit
def _run(scores_flat, deltas_flat, anchors_flat, idx_pad):
    mesh = plsc.VectorSubcoreMesh(core_axis_name="c", subcore_axis_name="s")
    f = functools.partial(
        pl.kernel,
        out_type=jax.ShapeDtypeStruct((5 * B * V,), jnp.float32),
        mesh=mesh,
        compiler_params=pltpu.CompilerParams(needs_layout_passes=False),
        scratch_types=[
            pltpu.VMEM((IDX_ROWS, CH), jnp.int32),
            pltpu.VMEM((ROWS,), jnp.int32),
            pltpu.VMEM((4 * ROWS,), jnp.int32),
            pltpu.VMEM((4 * NPW,), jnp.int32),
            pltpu.VMEM((ROWS,), jnp.float32),
            pltpu.VMEM((4 * ROWS,), jnp.float32),
            pltpu.VMEM((4 * NPW,), jnp.float32),
            pltpu.VMEM((5 * ROWS,), jnp.float32),
            pltpu.SemaphoreType.DMA,
        ],
    )(_body)
    return f(scores_flat, deltas_flat, anchors_flat, idx_pad)


def kernel(scores, deltas, anchor_boxes, valid_indices):
    vi = valid_indices.astype(jnp.int32)
    idx_pad = jnp.zeros((VP,), jnp.int32).at[:V].set(vi)
    out = _run(scores.reshape(B * AB),
               deltas.transpose(0, 2, 1).reshape(B * 4 * AB),
               anchor_boxes.T.reshape(4 * AB),
               idx_pad)
    return out.reshape(5, B, V).transpose(1, 2, 0)


# speedup vs baseline: 1.0115x; 1.0115x over previous
import functools

import jax
import jax.numpy as jnp
from jax import lax
from jax.experimental import pallas as pl
from jax.experimental.pallas import tpu as pltpu
from jax.experimental.pallas import tpu_sc as plsc

B = 16
AB = 20000
V = 12000
NW = 32
NPW = 384
VP = NW * NPW
NTAIL = V - (NW - 1) * NPW
CH = 128
IDX_ROWS = NPW // CH
ROWS = B * NPW
DROWS = 4 * B * IDX_ROWS
NCHUNK = NPW // 16


def _body(scores_hbm, deltas_hbm, anchors_hbm, idx_hbm, out_hbm,
          idxv, gs, gd, ax, sb, db, ab, ob, sem):
    nc = 2
    w = lax.axis_index("s") * nc + lax.axis_index("c")
    vbase = w * NPW

    with jax.named_scope("ph_idx"):
        for k in range(IDX_ROWS):
            pltpu.sync_copy(idx_hbm.at[pl.ds(vbase + k * CH, CH)], idxv.at[k])

    def build(c, _):
        r = c // 8
        co = r * CH + (c % 8) * 16
        vch = idxv[r, pl.ds((c % 8) * 16, 16)]
        ax[pl.ds(co, 16)] = vch
        ax[pl.ds(NPW + co, 16)] = vch + AB
        ax[pl.ds(2 * NPW + co, 16)] = vch + 2 * AB
        ax[pl.ds(3 * NPW + co, 16)] = vch + 3 * AB
        for b in range(B):
            p = (3 * b) * CH + co
            gs[pl.ds(p, 16)] = vch + b * AB
            vb = vch + b * (4 * AB)
            p4 = (3 * b * 4) * CH + co
            gd[pl.ds(p4, 16)] = vb
            gd[pl.ds(p4 + 3 * CH, 16)] = vb + AB
            gd[pl.ds(p4 + 6 * CH, 16)] = vb + 2 * AB
            gd[pl.ds(p4 + 9 * CH, 16)] = vb + 3 * AB
        return _
    with jax.named_scope("ph_build"):
        lax.fori_loop(0, NCHUNK, build, None)

    with jax.named_scope("ph_fire"):
        def fire(i, _):
            ksl = pl.ds(i * 2048, 2048)
            pltpu.async_copy(deltas_hbm.at[gd.at[ksl]], db.at[ksl], sem)
            return _
        lax.fori_loop(0, (4 * ROWS) // 2048, fire, None)

        def fire_s(i, _):
            ksl = pl.ds(i * 2048, 2048)
            pltpu.async_copy(scores_hbm.at[gs.at[ksl]], sb.at[ksl], sem)
            return _
        lax.fori_loop(0, ROWS // 2048, fire_s, None)
        pltpu.async_copy(anchors_hbm.at[ax], ab, sem)

    with jax.named_scope("ph_drain"):
        pltpu.make_async_copy(scores_hbm.at[pl.ds(0, ROWS)], sb, sem).wait()
        pltpu.make_async_copy(deltas_hbm.at[pl.ds(0, 4 * ROWS)], db, sem).wait()
        pltpu.make_async_copy(scores_hbm.at[pl.ds(0, 4 * NPW)], ab, sem).wait()

    def compute(c, _):
        x = ab[pl.ds(c * 16, 16)]
        y = ab[pl.ds(NPW + c * 16, 16)]
        ww = ab[pl.ds(2 * NPW + c * 16, 16)]
        hh = ab[pl.ds(3 * NPW + c * 16, 16)]
        for b in range(B):
            p0 = b * NPW + c * 16
            sl = pl.ds(p0, 16)
            dbase = 4 * b * NPW + c * 16
            d0 = db[pl.ds(dbase, 16)]
            d1 = db[pl.ds(dbase + NPW, 16)]
            d2 = db[pl.ds(dbase + 2 * NPW, 16)]
            d3 = db[pl.ds(dbase + 3 * NPW, 16)]
            ob[sl] = sb[sl]
            ob[pl.ds(ROWS + p0, 16)] = x + d0 * ww
            ob[pl.ds(2 * ROWS + p0, 16)] = y + d1 * hh
            ob[pl.ds(3 * ROWS + p0, 16)] = ww * jnp.exp(d2)
            ob[pl.ds(4 * ROWS + p0, 16)] = hh * jnp.exp(d3)
        return _
    with jax.named_scope("ph_compute"):
        lax.fori_loop(0, NCHUNK, compute, None)

    with jax.named_scope("ph_wb"):
        @pl.when(w < NW - 1)
        def _full():
            def wb(j, _):
                pltpu.async_copy(ob.at[pl.ds(j * NPW, NPW)],
                                 out_hbm.at[pl.ds(j * V + vbase, NPW)], sem)
                return _
            lax.fori_loop(0, 5 * B, wb, None)
            pltpu.make_async_copy(out_hbm.at[pl.ds(0, 5 * ROWS)], ob, sem).wait()

        @pl.when(w == NW - 1)
        def _tail():
            def wb(j, _):
                pltpu.async_copy(ob.at[pl.ds(j * NPW, NTAIL)],
                                 out_hbm.at[pl.ds(j * V + vbase, NTAIL)], sem)
                return _
            lax.fori_loop(0, 5 * B, wb, None)
            for j in range(5 * B):
                pltpu.make_async_copy(out_hbm.at[pl.ds(0, NTAIL)],
                                      ob.at[pl.ds(j * NPW, NTAIL)], sem).wait()


@jax.jit
def _run(scores_flat, deltas_flat, anchors_flat, idx_pad):
    mesh = plsc.VectorSubcoreMesh(core_axis_name="c", subcore_axis_name="s")
    f = functools.partial(
        pl.kernel,
        out_type=jax.ShapeDtypeStruct((5 * B * V,), jnp.float32),
        mesh=mesh,
        compiler_params=pltpu.CompilerParams(needs_layout_passes=False),
        scratch_types=[
            pltpu.VMEM((IDX_ROWS, CH), jnp.int32),
            pltpu.VMEM((ROWS,), jnp.int32),
            pltpu.VMEM((4 * ROWS,), jnp.int32),
            pltpu.VMEM((4 * NPW,), jnp.int32),
            pltpu.VMEM((ROWS,), jnp.float32),
            pltpu.VMEM((4 * ROWS,), jnp.float32),
            pltpu.VMEM((4 * NPW,), jnp.float32),
            pltpu.VMEM((5 * ROWS,), jnp.float32),
            pltpu.SemaphoreType.DMA,
        ],
    )(_body)
    return f(scores_flat, deltas_flat, anchors_flat, idx_pad)


def kernel(scores, deltas, anchor_boxes, valid_indices):
    vi = valid_indices.astype(jnp.int32)
    idx_pad = jnp.zeros((VP,), jnp.int32).at[:V].set(vi)
    out = _run(scores.reshape(B * AB),
               deltas.transpose(0, 2, 1).reshape(B * 4 * AB),
               anchor_boxes.T.reshape(4 * AB),
               idx_pad)
    return out.reshape(5, B, V).transpose(1, 2, 0)


# speedup vs baseline: 2.4916x; 2.4634x over previous
import functools

import jax
import jax.numpy as jnp
from jax import lax
from jax.experimental import pallas as pl
from jax.experimental.pallas import tpu as pltpu
from jax.experimental.pallas import tpu_sc as plsc

B = 16
AB = 20000
PAD = 20096
V = 12000
NW = 32
NPW = 384
VP = NW * NPW
NTAIL = V - (NW - 1) * NPW
CH = 128
IDX_ROWS = NPW // CH
ROWS = B * NPW
NCHUNK = NPW // 16
WIN = 1024
ABW = PAD - WIN


def _body(scores_hbm, deltas_hbm, anchors_hbm, idx_hbm, out_hbm,
          idxv, dwin, swin, awin, ob, sem):
    nc = 2
    w = lax.axis_index("s") * nc + lax.axis_index("c")
    vbase = w * NPW

    for k in range(IDX_ROWS):
        pltpu.sync_copy(idx_hbm.at[pl.ds(vbase + k * CH, CH)], idxv.at[k])

    lo = jnp.min(idxv[0, pl.ds(0, 16)])
    hi = jnp.max(idxv[IDX_ROWS - 1, pl.ds(CH - 16, 16)])
    lo128 = jnp.bitwise_and(lo, -128)
    npass = (hi - lo128) // WIN + 1

    iota = lax.iota(jnp.int32, 16)
    zz = jnp.zeros((16,), jnp.int32)

    def do_pass(p, _):
        base = pl.multiple_of(jnp.minimum(lo128 + p * WIN, ABW), CH)
        wsl = pl.ds(base, WIN)
        pltpu.async_copy(scores_hbm.at[:, wsl], swin, sem)
        pltpu.async_copy(anchors_hbm.at[:, wsl], awin, sem)
        pltpu.async_copy(deltas_hbm.at[:, wsl], dwin, sem)
        zsl = pl.ds(0, WIN)
        pltpu.make_async_copy(scores_hbm.at[:, zsl], swin, sem).wait()
        pltpu.make_async_copy(anchors_hbm.at[:, zsl], awin, sem).wait()
        pltpu.make_async_copy(deltas_hbm.at[:, zsl], dwin, sem).wait()

        def compute(c, _c):
            r = c // 8
            co = (c % 8) * 16
            lc = idxv[r, pl.ds(co, 16)] - base
            m = (lc >= 0) & (lc < WIN)
            x = plsc.load_gather(awin, [zz, lc], mask=m)
            y = plsc.load_gather(awin, [zz + 1, lc], mask=m)
            ww = plsc.load_gather(awin, [zz + 2, lc], mask=m)
            hh = plsc.load_gather(awin, [zz + 3, lc], mask=m)
            for b in range(B):
                s = plsc.load_gather(swin, [zz + b, lc], mask=m)
                d0 = plsc.load_gather(dwin, [zz + 4 * b, lc], mask=m)
                d1 = plsc.load_gather(dwin, [zz + (4 * b + 1), lc], mask=m)
                d2 = plsc.load_gather(dwin, [zz + (4 * b + 2), lc], mask=m)
                d3 = plsc.load_gather(dwin, [zz + (4 * b + 3), lc], mask=m)
                rowi = (b * NPW + c * 16) + iota
                plsc.store_scatter(ob, [rowi], s, mask=m)
                plsc.store_scatter(ob, [rowi + ROWS], x + d0 * ww, mask=m)
                plsc.store_scatter(ob, [rowi + 2 * ROWS], y + d1 * hh, mask=m)
                plsc.store_scatter(ob, [rowi + 3 * ROWS],
                                   ww * jnp.exp(d2), mask=m)
                plsc.store_scatter(ob, [rowi + 4 * ROWS],
                                   hh * jnp.exp(d3), mask=m)
            return _c
        lax.fori_loop(0, NCHUNK, compute, None)
        return _
    lax.fori_loop(0, npass, do_pass, None)

    @pl.when(w < NW - 1)
    def _full():
        def wb(j, _):
            pltpu.async_copy(ob.at[pl.ds(j * NPW, NPW)],
                             out_hbm.at[pl.ds(j * V + vbase, NPW)], sem)
            return _
        lax.fori_loop(0, 5 * B, wb, None)
        pltpu.make_async_copy(out_hbm.at[pl.ds(0, 5 * ROWS)], ob, sem).wait()

    @pl.when(w == NW - 1)
    def _tail():
        def wb(j, _):
            pltpu.async_copy(ob.at[pl.ds(j * NPW, NTAIL)],
                             out_hbm.at[pl.ds(j * V + vbase, NTAIL)], sem)
            return _
        lax.fori_loop(0, 5 * B, wb, None)
        for j in range(5 * B):
            pltpu.make_async_copy(out_hbm.at[pl.ds(0, NTAIL)],
                                  ob.at[pl.ds(j * NPW, NTAIL)], sem).wait()


@jax.jit
def _run(scores2d, deltas2d, anchors2d, idx_pad):
    mesh = plsc.VectorSubcoreMesh(core_axis_name="c", subcore_axis_name="s")
    f = functools.partial(
        pl.kernel,
        out_type=jax.ShapeDtypeStruct((5 * B * V,), jnp.float32),
        mesh=mesh,
        compiler_params=pltpu.CompilerParams(needs_layout_passes=False),
        scratch_types=[
            pltpu.VMEM((IDX_ROWS, CH), jnp.int32),
            pltpu.VMEM((4 * B, WIN), jnp.float32),
            pltpu.VMEM((B, WIN), jnp.float32),
            pltpu.VMEM((4, WIN), jnp.float32),
            pltpu.VMEM((5 * ROWS,), jnp.float32),
            pltpu.SemaphoreType.DMA,
        ],
    )(_body)
    return f(scores2d, deltas2d, anchors2d, idx_pad)


def kernel(scores, deltas, anchor_boxes, valid_indices):
    vi = valid_indices.astype(jnp.int32)
    idx_pad = jnp.pad(vi, (0, VP - V), mode="edge")
    pad_n = PAD - AB
    out = _run(jnp.pad(scores, ((0, 0), (0, pad_n))),
               jnp.pad(deltas.transpose(0, 2, 1).reshape(4 * B, AB),
                       ((0, 0), (0, pad_n))),
               jnp.pad(anchor_boxes.T, ((0, 0), (0, pad_n))),
               idx_pad)
    return out.reshape(5, B, V).transpose(1, 2, 0)


# speedup vs baseline: 2.4930x; 1.0005x over previous
import functools

import jax
import jax.numpy as jnp
from jax import lax
from jax.experimental import pallas as pl
from jax.experimental.pallas import tpu as pltpu
from jax.experimental.pallas import tpu_sc as plsc

B = 16
AB = 20000
PAD = 20096
V = 12000
NW = 32
NPW = 384
VP = NW * NPW
NTAIL = V - (NW - 1) * NPW
CH = 128
IDX_ROWS = NPW // CH
ROWS = B * NPW
NCHUNK = NPW // 16
WIN = 1024
ABW = PAD - WIN


def _body(scores_hbm, deltas_hbm, anchors_hbm, idx_hbm, out_hbm,
          idxv, dwin, swin, awin, ob, sem):
    nc = 2
    w = lax.axis_index("s") * nc + lax.axis_index("c")
    vbase = w * NPW

    for k in range(IDX_ROWS):
        pltpu.sync_copy(idx_hbm.at[pl.ds(vbase + k * CH, CH)], idxv.at[k])

    lo = jnp.min(idxv[0, pl.ds(0, 16)])
    hi = jnp.max(idxv[IDX_ROWS - 1, pl.ds(CH - 16, 16)])
    lo128 = jnp.bitwise_and(lo, -128)
    npass = (hi - lo128) // WIN + 1

    iota = lax.iota(jnp.int32, 16)
    zz = jnp.zeros((16,), jnp.int32)

    def do_pass(p, _):
        base = pl.multiple_of(jnp.minimum(lo128 + p * WIN, ABW), CH)
        wsl = pl.ds(base, WIN)
        hb2 = 2 * B
        pltpu.async_copy(scores_hbm.at[:, wsl], swin, sem)
        pltpu.async_copy(anchors_hbm.at[:, wsl], awin, sem)
        pltpu.async_copy(deltas_hbm.at[pl.ds(0, hb2), wsl],
                         dwin.at[pl.ds(0, hb2)], sem)
        zsl = pl.ds(0, WIN)
        pltpu.make_async_copy(scores_hbm.at[:, zsl], swin, sem).wait()
        pltpu.make_async_copy(anchors_hbm.at[:, zsl], awin, sem).wait()
        pltpu.make_async_copy(deltas_hbm.at[pl.ds(0, hb2), zsl],
                              dwin.at[pl.ds(0, hb2)], sem).wait()
        pltpu.async_copy(deltas_hbm.at[pl.ds(hb2, hb2), wsl],
                         dwin.at[pl.ds(hb2, hb2)], sem)

        def mk_compute(b0, b1):
            def compute(c, _c):
                r = c // 8
                co = (c % 8) * 16
                lc = idxv[r, pl.ds(co, 16)] - base
                m = (lc >= 0) & (lc < WIN)
                x = plsc.load_gather(awin, [zz, lc], mask=m)
                y = plsc.load_gather(awin, [zz + 1, lc], mask=m)
                ww = plsc.load_gather(awin, [zz + 2, lc], mask=m)
                hh = plsc.load_gather(awin, [zz + 3, lc], mask=m)
                for b in range(b0, b1):
                    s = plsc.load_gather(swin, [zz + b, lc], mask=m)
                    d0 = plsc.load_gather(dwin, [zz + 4 * b, lc], mask=m)
                    d1 = plsc.load_gather(dwin, [zz + (4 * b + 1), lc],
                                          mask=m)
                    d2 = plsc.load_gather(dwin, [zz + (4 * b + 2), lc],
                                          mask=m)
                    d3 = plsc.load_gather(dwin, [zz + (4 * b + 3), lc],
                                          mask=m)
                    rowi = (b * NPW + c * 16) + iota
                    plsc.store_scatter(ob, [rowi], s, mask=m)
                    plsc.store_scatter(ob, [rowi + ROWS],
                                       x + d0 * ww, mask=m)
                    plsc.store_scatter(ob, [rowi + 2 * ROWS],
                                       y + d1 * hh, mask=m)
                    plsc.store_scatter(ob, [rowi + 3 * ROWS],
                                       ww * jnp.exp(d2), mask=m)
                    plsc.store_scatter(ob, [rowi + 4 * ROWS],
                                       hh * jnp.exp(d3), mask=m)
                return _c
            return compute
        lax.fori_loop(0, NCHUNK, mk_compute(0, B // 2), None)
        pltpu.make_async_copy(deltas_hbm.at[pl.ds(hb2, hb2), zsl],
                              dwin.at[pl.ds(hb2, hb2)], sem).wait()
        lax.fori_loop(0, NCHUNK, mk_compute(B // 2, B), None)
        return _
    lax.fori_loop(0, npass, do_pass, None)

    @pl.when(w < NW - 1)
    def _full():
        def wb(j, _):
            pltpu.async_copy(ob.at[pl.ds(j * NPW, NPW)],
                             out_hbm.at[pl.ds(j * V + vbase, NPW)], sem)
            return _
        lax.fori_loop(0, 5 * B, wb, None)
        pltpu.make_async_copy(out_hbm.at[pl.ds(0, 5 * ROWS)], ob, sem).wait()

    @pl.when(w == NW - 1)
    def _tail():
        def wb(j, _):
            pltpu.async_copy(ob.at[pl.ds(j * NPW, NTAIL)],
                             out_hbm.at[pl.ds(j * V + vbase, NTAIL)], sem)
            return _
        lax.fori_loop(0, 5 * B, wb, None)
        for j in range(5 * B):
            pltpu.make_async_copy(out_hbm.at[pl.ds(0, NTAIL)],
                                  ob.at[pl.ds(j * NPW, NTAIL)], sem).wait()


@jax.jit
def _run(scores2d, deltas2d, anchors2d, idx_pad):
    mesh = plsc.VectorSubcoreMesh(core_axis_name="c", subcore_axis_name="s")
    f = functools.partial(
        pl.kernel,
        out_type=jax.ShapeDtypeStruct((5 * B * V,), jnp.float32),
        mesh=mesh,
        compiler_params=pltpu.CompilerParams(needs_layout_passes=False),
        scratch_types=[
            pltpu.VMEM((IDX_ROWS, CH), jnp.int32),
            pltpu.VMEM((4 * B, WIN), jnp.float32),
            pltpu.VMEM((B, WIN), jnp.float32),
            pltpu.VMEM((4, WIN), jnp.float32),
            pltpu.VMEM((5 * ROWS,), jnp.float32),
            pltpu.SemaphoreType.DMA,
        ],
    )(_body)
    return f(scores2d, deltas2d, anchors2d, idx_pad)


def kernel(scores, deltas, anchor_boxes, valid_indices):
    vi = valid_indices.astype(jnp.int32)
    idx_pad = jnp.pad(vi, (0, VP - V), mode="edge")
    pad_n = PAD - AB
    out = _run(jnp.pad(scores, ((0, 0), (0, pad_n))),
               jnp.pad(deltas.transpose(0, 2, 1).reshape(4 * B, AB),
                       ((0, 0), (0, pad_n))),
               jnp.pad(anchor_boxes.T, ((0, 0), (0, pad_n))),
               idx_pad)
    return out.reshape(5, B, V).transpose(1, 2, 0)
